# Initial kernel scaffold; baseline (speedup 1.0000x reference)
import jax
import jax.numpy as jnp
from jax.experimental import pallas as pl


def kernel(x, edge_index, edge_weight, W, b):
    raise NotImplementedError("write your pallas kernel here")


# speedup vs baseline: 6.8653x; 6.8653x over previous
import functools

import jax
import jax.numpy as jnp
from jax import lax
from jax.experimental import pallas as pl
from jax.experimental.pallas import tpu as pltpu
from jax.experimental.pallas import tpu_sc as plsc

N = 10000
D = 128
E = 320000

NUM_CORES = 2
NUM_SUBCORES = 16
NUM_TILES = NUM_CORES * NUM_SUBCORES
EDGES_PER_TILE = E // NUM_TILES
CHUNK = 80
CHUNKS_PER_TILE = EDGES_PER_TILE // CHUNK
NPAD = 10240
ROWS_PER_TILE = NPAD // NUM_SUBCORES
LANES = 16
D_BLKS = D // LANES


def _sc_aggregate(x, src, dst3, ew):
    mesh = plsc.VectorSubcoreMesh(core_axis_name="c", subcore_axis_name="s")

    @functools.partial(
        pl.kernel,
        out_type=jax.ShapeDtypeStruct((NUM_CORES, NPAD, D), jnp.float32),
        mesh=mesh,
        scratch_types=[
            pltpu.VMEM((EDGES_PER_TILE,), jnp.int32),
            pltpu.VMEM((CHUNKS_PER_TILE, CHUNK), jnp.int32),
            pltpu.VMEM((EDGES_PER_TILE,), jnp.float32),
            pltpu.VMEM((CHUNK, D), jnp.float32),
            pltpu.VMEM_SHARED((NPAD, D), jnp.float32),
            pltpu.SemaphoreType.DMA,
        ],
    )
    def k(x_hbm, src_hbm, dst_hbm, ew_hbm, out_hbm, src_v, dst_v, ew_v, rows_v, acc_sh, sem):
        c = lax.axis_index("c")
        s = lax.axis_index("s")
        wid = s * NUM_CORES + c

        def zrow(i, carry):
            for d in range(D_BLKS):
                rows_v[i, pl.ds(d * LANES, LANES)] = jnp.zeros((LANES,), jnp.float32)
            return carry

        lax.fori_loop(0, CHUNK, zrow, 0)
        row0 = s * ROWS_PER_TILE
        for r in range(ROWS_PER_TILE // CHUNK):
            pltpu.sync_copy(rows_v, acc_sh.at[pl.ds(row0 + r * CHUNK, CHUNK)])

        e0 = wid * EDGES_PER_TILE
        pltpu.sync_copy(src_hbm.at[pl.ds(e0, EDGES_PER_TILE)], src_v)
        pltpu.sync_copy(dst_hbm.at[wid], dst_v)
        pltpu.sync_copy(ew_hbm.at[pl.ds(e0, EDGES_PER_TILE)], ew_v)

        plsc.subcore_barrier()

        def chunk_body(j, carry):
            base = j * CHUNK
            pltpu.async_copy(
                x_hbm.at[src_v.at[pl.ds(base, CHUNK)]], rows_v, sem
            ).wait()

            def scale_group(g, carry2):
                wv = ew_v[pl.ds(base + g * LANES, LANES)]
                for e in range(LANES):
                    w = jnp.broadcast_to(wv[e], (LANES,))
                    row = g * LANES + e
                    for d in range(D_BLKS):
                        rows_v[row, pl.ds(d * LANES, LANES)] = (
                            rows_v[row, pl.ds(d * LANES, LANES)] * w
                        )
                return carry2

            lax.fori_loop(0, CHUNK // LANES, scale_group, 0)
            pltpu.sync_copy(rows_v, acc_sh.at[dst_v.at[j]], add=True)
            return carry

        lax.fori_loop(0, CHUNKS_PER_TILE, chunk_body, 0)

        plsc.subcore_barrier()

        pltpu.sync_copy(
            acc_sh.at[pl.ds(row0, ROWS_PER_TILE)],
            out_hbm.at[c, pl.ds(row0, ROWS_PER_TILE)],
        )

    return k(x, src, dst3, ew)


def _tc_finish(parts, W, b2):
    blk = 1000

    def body(p_ref, w_ref, b_ref, o_ref):
        acc = p_ref[0] + p_ref[1]
        h = jnp.dot(acc, w_ref[...], preferred_element_type=jnp.float32)
        o_ref[...] = jnp.maximum(h + b_ref[...], 0.0)

    return pl.pallas_call(
        body,
        grid=(N // blk,),
        in_specs=[
            pl.BlockSpec((NUM_CORES, blk, D), lambda i: (0, i, 0)),
            pl.BlockSpec((D, D), lambda i: (0, 0)),
            pl.BlockSpec((1, D), lambda i: (0, 0)),
        ],
        out_specs=pl.BlockSpec((blk, D), lambda i: (i, 0)),
        out_shape=jax.ShapeDtypeStruct((N, D), jnp.float32),
    )(parts, W, b2)


def kernel(x, edge_index, edge_weight, W, b):
    ei = edge_index.astype(jnp.int32)
    src = ei[0]
    dst3 = ei[1].reshape(NUM_TILES, CHUNKS_PER_TILE, CHUNK)
    parts = _sc_aggregate(x, src, dst3, edge_weight)
    return _tc_finish(parts, W, b.reshape(1, D))


# speedup vs baseline: 12.7558x; 1.8580x over previous
import functools

import jax
import jax.numpy as jnp
from jax import lax
from jax.experimental import pallas as pl
from jax.experimental.pallas import tpu as pltpu
from jax.experimental.pallas import tpu_sc as plsc

N = 10000
D = 128
E = 320000

NUM_CORES = 2
NUM_SUBCORES = 16
NUM_TILES = NUM_CORES * NUM_SUBCORES
EDGES_PER_TILE = E // NUM_TILES
CHUNK = 80
CHUNKS_PER_TILE = EDGES_PER_TILE // CHUNK
RING = 3
LOOK = 1
NPAD = 10240
ROWS_PER_TILE = NPAD // NUM_SUBCORES
LANES = 16
D_BLKS = D // LANES


def _sc_aggregate(x, src, dst3, ew):
    mesh = plsc.VectorSubcoreMesh(core_axis_name="c", subcore_axis_name="s")

    @functools.partial(
        pl.kernel,
        out_type=jax.ShapeDtypeStruct((NUM_CORES, NPAD, D), jnp.float32),
        mesh=mesh,
        scratch_types=[
            pltpu.VMEM((EDGES_PER_TILE,), jnp.int32),
            pltpu.VMEM((RING, CHUNK), jnp.int32),
            pltpu.VMEM((RING, CHUNK), jnp.float32),
            pltpu.VMEM((RING, CHUNK, D), jnp.float32),
            pltpu.VMEM_SHARED((NPAD, D), jnp.float32),
            pltpu.SemaphoreType.DMA((RING,)),
            pltpu.SemaphoreType.DMA((RING,)),
            pltpu.SemaphoreType.DMA((RING,)),
            pltpu.SemaphoreType.DMA((RING,)),
        ],
    )
    def k(x_hbm, src_hbm, dst_hbm, ew_hbm, out_hbm, src_v, dst_v, ew_v, rows_v,
          acc_sh, semg, sems, semd, seme):
        c = lax.axis_index("c")
        s = lax.axis_index("s")
        wid = s * NUM_CORES + c

        def zrow(i, carry):
            for d in range(D_BLKS):
                rows_v[0, i, pl.ds(d * LANES, LANES)] = jnp.zeros((LANES,), jnp.float32)
            return carry

        lax.fori_loop(0, CHUNK, zrow, 0)
        row0 = s * ROWS_PER_TILE
        for r in range(ROWS_PER_TILE // CHUNK):
            pltpu.sync_copy(rows_v.at[0], acc_sh.at[pl.ds(row0 + r * CHUNK, CHUNK)])

        e0 = wid * EDGES_PER_TILE
        pltpu.sync_copy(src_hbm.at[pl.ds(e0, EDGES_PER_TILE)], src_v)

        plsc.subcore_barrier()

        def start_gather(j, p):
            pltpu.async_copy(
                x_hbm.at[src_v.at[pl.ds(j * CHUNK, CHUNK)]],
                rows_v.at[p],
                semg.at[p],
            )
            pltpu.async_copy(
                dst_hbm.at[pl.ds(e0 + j * CHUNK, CHUNK)], dst_v.at[p], semd.at[p]
            )
            pltpu.async_copy(
                ew_hbm.at[pl.ds(e0 + j * CHUNK, CHUNK)], ew_v.at[p], seme.at[p]
            )

        def wait_gather(j, p):
            pltpu.make_async_copy(
                x_hbm.at[src_v.at[pl.ds(j * CHUNK, CHUNK)]],
                rows_v.at[p],
                semg.at[p],
            ).wait()
            pltpu.make_async_copy(
                dst_hbm.at[pl.ds(e0 + j * CHUNK, CHUNK)], dst_v.at[p], semd.at[p]
            ).wait()
            pltpu.make_async_copy(
                ew_hbm.at[pl.ds(e0 + j * CHUNK, CHUNK)], ew_v.at[p], seme.at[p]
            ).wait()

        def start_scatter(j, p):
            pltpu.async_copy(
                rows_v.at[p], acc_sh.at[dst_v.at[p]], sems.at[p], add=True
            )

        def wait_scatter(p):
            pltpu.make_async_copy(
                rows_v.at[p], acc_sh.at[dst_v.at[p]], sems.at[p]
            ).wait()

        for k0 in range(LOOK):
            start_gather(k0, k0)

        def chunk_body(j, p):
            nj = j + LOOK
            np_ = (p + LOOK) % RING

            @pl.when(nj < CHUNKS_PER_TILE)
            def _():
                @pl.when(nj >= RING)
                def _():
                    wait_scatter(np_)

                start_gather(nj, np_)

            wait_gather(j, p)

            def scale_group(g, carry2):
                wv = ew_v[p, pl.ds(g * LANES, LANES)]
                for e in range(LANES):
                    w = jnp.broadcast_to(wv[e], (LANES,))
                    row = g * LANES + e
                    for d in range(D_BLKS):
                        rows_v[p, row, pl.ds(d * LANES, LANES)] = (
                            rows_v[p, row, pl.ds(d * LANES, LANES)] * w
                        )
                return carry2

            lax.fori_loop(0, CHUNK // LANES, scale_group, 0)
            start_scatter(j, p)

        def ring_step(t, carry):
            for p in range(RING):
                chunk_body(t * RING + p, p)
            return carry

        full = (CHUNKS_PER_TILE // RING) * RING
        lax.fori_loop(0, CHUNKS_PER_TILE // RING, ring_step, 0)
        for j in range(full, CHUNKS_PER_TILE):
            chunk_body(jnp.int32(j), j % RING)

        for p in range(RING):
            wait_scatter((CHUNKS_PER_TILE - RING + p) % RING)

        plsc.subcore_barrier()

        pltpu.sync_copy(
            acc_sh.at[pl.ds(row0, ROWS_PER_TILE)],
            out_hbm.at[c, pl.ds(row0, ROWS_PER_TILE)],
        )

    return k(x, src, dst3, ew)


def _tc_finish(parts, W, b2):
    blk = 1000

    def body(p_ref, w_ref, b_ref, o_ref):
        acc = p_ref[0] + p_ref[1]
        h = jnp.dot(acc, w_ref[...], preferred_element_type=jnp.float32)
        o_ref[...] = jnp.maximum(h + b_ref[...], 0.0)

    return pl.pallas_call(
        body,
        grid=(N // blk,),
        in_specs=[
            pl.BlockSpec((NUM_CORES, blk, D), lambda i: (0, i, 0)),
            pl.BlockSpec((D, D), lambda i: (0, 0)),
            pl.BlockSpec((1, D), lambda i: (0, 0)),
        ],
        out_specs=pl.BlockSpec((blk, D), lambda i: (i, 0)),
        out_shape=jax.ShapeDtypeStruct((N, D), jnp.float32),
    )(parts, W, b2)


def kernel(x, edge_index, edge_weight, W, b):
    ei = edge_index.astype(jnp.int32)
    src = ei[0]
    parts = _sc_aggregate(x, src, ei[1], edge_weight)
    return _tc_finish(parts, W, b.reshape(1, D))
